# baseline (speedup 1.0000x reference)
import functools

import jax
import jax.numpy as jnp
from jax import lax
from jax.experimental import pallas as pl
from jax.experimental.pallas import tpu as pltpu
from jax.experimental.pallas import tpu_sc as plsc

L = 16
NC = 2
NS = 16
NW = NC * NS
B = 16384
F = 64
BPW = B // NW
IDX_CHUNK = 128
NCHUNK = BPW // IDX_CHUNK


def _make_kernel():
    mesh = plsc.VectorSubcoreMesh(core_axis_name="c", subcore_axis_name="s")

    @functools.partial(
        pl.kernel,
        mesh=mesh,
        compiler_params=pltpu.CompilerParams(
            needs_layout_passes=False, use_tc_tiling_on_sc=False),
        out_type=jax.ShapeDtypeStruct((B,), jnp.float32),
        scratch_types=[
            pltpu.VMEM((NCHUNK, IDX_CHUNK), jnp.int32),
            pltpu.VMEM((NCHUNK, IDX_CHUNK), jnp.int32),
            pltpu.VMEM((BPW, F), jnp.float32),
            pltpu.VMEM((BPW, F), jnp.float32),
            pltpu.VMEM((L * F,), jnp.float32),
            pltpu.VMEM((BPW,), jnp.float32),
            pltpu.SemaphoreType.DMA,
        ],
    )
    def kern(user_hbm, book_hbm, uf_hbm, bf_hbm, out_hbm,
             uidx_v, bidx_v, urows_v, brows_v, trans_v, out_v, sem):
        wid = lax.axis_index("s") * NC + lax.axis_index("c")
        base = wid * BPW

        pltpu.sync_copy(user_hbm.at[wid], uidx_v)
        pltpu.sync_copy(book_hbm.at[wid], bidx_v)

        copies = []
        for i in range(NCHUNK):
            copies.append(pltpu.async_copy(
                uf_hbm.at[uidx_v.at[i]],
                urows_v.at[pl.ds(i * IDX_CHUNK, IDX_CHUNK)], sem))
            copies.append(pltpu.async_copy(
                bf_hbm.at[bidx_v.at[i]],
                brows_v.at[pl.ds(i * IDX_CHUNK, IDX_CHUNK)], sem))
        for cp in copies:
            cp.wait()

        lanes = lax.iota(jnp.int32, L)
        cbases = [c * L * L + lanes * L for c in range(F // L)]

        def group_body(g, carry):
            row0 = g * L

            def row_body(r, _):
                for c in range(F // L):
                    u = urows_v[row0 + r, pl.ds(c * L, L)]
                    b = brows_v[row0 + r, pl.ds(c * L, L)]
                    plsc.store_scatter(trans_v, [cbases[c] + r], u * b)
                return _

            lax.fori_loop(0, L, row_body, 0, unroll=4)

            def feat_body(f, acc):
                return acc + trans_v[pl.ds(f * L, L)]

            acc = lax.fori_loop(0, F, feat_body, jnp.zeros((L,), jnp.float32),
                                unroll=8)
            out_v[pl.ds(row0, L)] = acc
            return carry

        lax.fori_loop(0, BPW // L, group_body, 0)

        pltpu.sync_copy(out_v, out_hbm.at[pl.ds(base, BPW)])

    return kern


_kernel = _make_kernel()


@jax.jit
def kernel(user, book, user_factors, book_factors):
    user_r = user.astype(jnp.int32).reshape(NW, NCHUNK, IDX_CHUNK)
    book_r = book.astype(jnp.int32).reshape(NW, NCHUNK, IDX_CHUNK)
    return _kernel(user_r, book_r, user_factors, book_factors)
